# baseline (speedup 1.0000x reference)
import functools

import jax
import jax.numpy as jnp
from jax import lax
from jax.experimental import pallas as pl
from jax.experimental.pallas import tpu as pltpu
from jax.experimental.pallas import tpu_sc as plsc

_CB = 80
_GPB = 2
_NBUF = 4


def _embed_lookup(idx2d, table, B, V, D):
    info = plsc.get_sparse_core_info()
    NC, NS = info.num_cores, info.num_subcores
    NW = NC * NS
    b_per_w = B // NW
    rows_per_chunk = _GPB * _CB
    n_idx_rows = b_per_w // _CB
    n_chunks = b_per_w // rows_per_chunk
    n_outer = n_chunks // _NBUF
    n_tail = n_chunks - n_outer * _NBUF
    mesh = plsc.VectorSubcoreMesh(core_axis_name="c", subcore_axis_name="s")

    @functools.partial(
        pl.kernel,
        mesh=mesh,
        out_type=jax.ShapeDtypeStruct((B, D), jnp.float32),
        scratch_types=[
            pltpu.VMEM((_GPB * n_chunks, _CB), jnp.int32),
            pltpu.VMEM((_NBUF, rows_per_chunk, D), jnp.float32),
            pltpu.VMEM_SHARED((V, D), jnp.float32),
            pltpu.SemaphoreType.DMA,
            pltpu.SemaphoreType.DMA,
            pltpu.SemaphoreType.DMA,
            pltpu.SemaphoreType.DMA,
        ],
    )
    def emb(idx_hbm, table_hbm, out_hbm, idx_all, rows, table_sh, s0, s1, s2, s3):
        sems = [s0, s1, s2, s3]
        sid = lax.axis_index("s")
        wid = sid * NC + lax.axis_index("c")
        base = wid * b_per_w

        rows_per_stager = V // 5

        @pl.when(sid < 5)
        def _():
            pltpu.sync_copy(
                table_hbm.at[pl.ds(sid * rows_per_stager, rows_per_stager)],
                table_sh.at[pl.ds(sid * rows_per_stager, rows_per_stager)],
            )

        pltpu.sync_copy(idx_hbm.at[pl.ds(wid * n_idx_rows, n_idx_rows)], idx_all)
        plsc.subcore_barrier()

        def start_chunk(g, b):
            for j in range(_GPB):
                pltpu.async_copy(
                    table_sh.at[idx_all.at[g * _GPB + j]],
                    rows.at[b].at[pl.ds(j * _CB, _CB)],
                    sems[b],
                )

        def wait_chunk(b):
            pltpu.make_async_copy(
                table_sh.at[idx_all.at[0]], rows.at[b], sems[b]
            ).wait()

        def write_chunk(g, b):
            pltpu.sync_copy(
                rows.at[b],
                out_hbm.at[pl.ds(base + g * rows_per_chunk, rows_per_chunk)],
            )

        for b in range(_NBUF):
            start_chunk(b, b)

        def round_body(k, carry):
            for b in range(_NBUF):
                g = k * _NBUF + b
                wait_chunk(b)
                write_chunk(g, b)

                @pl.when(g + _NBUF < n_chunks)
                def _():
                    start_chunk(g + _NBUF, b)

            return carry

        lax.fori_loop(0, n_outer, round_body, 0)

        for e in range(n_tail):
            g = n_outer * _NBUF + e
            b = g % _NBUF
            wait_chunk(b)
            write_chunk(g, b)

    return emb(idx2d, table)


def kernel(input_ids, embed_table):
    V, D = embed_table.shape
    B = input_ids.size
    idx2d = input_ids.reshape((B // _CB, _CB)).astype(jnp.int32)
    out = _embed_lookup(idx2d, embed_table, B, V, D)
    return out.reshape(input_ids.shape + (D,))


# speedup vs baseline: 1.0130x; 1.0130x over previous
import functools

import jax
import jax.numpy as jnp
from jax import lax
from jax.experimental import pallas as pl
from jax.experimental.pallas import tpu as pltpu
from jax.experimental.pallas import tpu_sc as plsc

_CB = 100
_GPB = 2
_NBUF = 3


def _embed_lookup(idx2d, table, B, V, D):
    info = plsc.get_sparse_core_info()
    NC, NS = info.num_cores, info.num_subcores
    NW = NC * NS
    b_per_w = B // NW
    rows_per_chunk = _GPB * _CB
    n_idx_rows = b_per_w // _CB
    n_chunks = b_per_w // rows_per_chunk
    n_outer = n_chunks // _NBUF
    n_tail = n_chunks - n_outer * _NBUF
    mesh = plsc.VectorSubcoreMesh(core_axis_name="c", subcore_axis_name="s")

    @functools.partial(
        pl.kernel,
        mesh=mesh,
        out_type=jax.ShapeDtypeStruct((B, D), jnp.float32),
        scratch_types=[
            pltpu.VMEM((_GPB * n_chunks, _CB), jnp.int32),
            pltpu.VMEM((_NBUF, rows_per_chunk, D), jnp.float32),
            pltpu.VMEM_SHARED((V, D), jnp.float32),
            pltpu.SemaphoreType.DMA,
            pltpu.SemaphoreType.DMA,
            pltpu.SemaphoreType.DMA,
        ],
    )
    def emb(idx_hbm, table_hbm, out_hbm, idx_all, rows, table_sh, s0, s1, s2):
        sems = [s0, s1, s2]
        sid = lax.axis_index("s")
        wid = sid * NC + lax.axis_index("c")
        base = wid * b_per_w

        rows_per_stager = V // 5

        @pl.when(sid < 5)
        def _():
            pltpu.sync_copy(
                table_hbm.at[pl.ds(sid * rows_per_stager, rows_per_stager)],
                table_sh.at[pl.ds(sid * rows_per_stager, rows_per_stager)],
            )

        pltpu.sync_copy(idx_hbm.at[pl.ds(wid * n_idx_rows, n_idx_rows)], idx_all)
        plsc.subcore_barrier()

        def start_chunk(g, b):
            for j in range(_GPB):
                pltpu.async_copy(
                    table_sh.at[idx_all.at[g * _GPB + j]],
                    rows.at[b].at[pl.ds(j * _CB, _CB)],
                    sems[b],
                )

        def wait_chunk(b):
            pltpu.make_async_copy(
                table_sh.at[idx_all.at[0]], rows.at[b], sems[b]
            ).wait()

        def write_chunk(g, b):
            pltpu.sync_copy(
                rows.at[b],
                out_hbm.at[pl.ds(base + g * rows_per_chunk, rows_per_chunk)],
            )

        for b in range(_NBUF):
            start_chunk(b, b)

        def round_body(k, carry):
            for b in range(_NBUF):
                g = k * _NBUF + b
                wait_chunk(b)
                write_chunk(g, b)

                @pl.when(g + _NBUF < n_chunks)
                def _():
                    start_chunk(g + _NBUF, b)

            return carry

        lax.fori_loop(0, n_outer, round_body, 0)

        for e in range(n_tail):
            g = n_outer * _NBUF + e
            b = g % _NBUF
            wait_chunk(b)
            write_chunk(g, b)

    return emb(idx2d, table)


def kernel(input_ids, embed_table):
    V, D = embed_table.shape
    B = input_ids.size
    idx2d = input_ids.reshape((B // _CB, _CB)).astype(jnp.int32)
    out = _embed_lookup(idx2d, embed_table, B, V, D)
    return out.reshape(input_ids.shape + (D,))
